# Initial kernel scaffold; baseline (speedup 1.0000x reference)
import jax
import jax.numpy as jnp
from jax.experimental import pallas as pl


def kernel(x, edge_index, feat_rel, W_src, W_dst, W_rel):
    raise NotImplementedError("write your pallas kernel here")


# speedup vs baseline: 1.0589x; 1.0589x over previous
import functools

import jax
import jax.numpy as jnp
from jax import lax
from jax.experimental import pallas as pl
from jax.experimental.pallas import tpu as pltpu

N = 10000
E = 320000
D_IN = 128
D_REL = 64
K = 8
D_OUT = 16
KD = K * D_OUT

N_PAD = 10240
_ROW_BLK = 512


def _dense_body(x_ref, fr_ref, wsrc_ref, wdst_ref, wrel_ref,
                feat_ref, el_ref, er_ref):
    x = x_ref[...]
    fs = jnp.dot(x, wsrc_ref[...], preferred_element_type=jnp.float32)
    fd = jnp.dot(x, wdst_ref[...], preferred_element_type=jnp.float32)
    attn = jnp.dot(fr_ref[...], wrel_ref[...],
                   preferred_element_type=jnp.float32)
    rr = lax.broadcasted_iota(jnp.int32, (2 * KD, KD), 0)
    cc = lax.broadcasted_iota(jnp.int32, (2 * KD, KD), 1)
    sel_l = jnp.where(rr == (cc // D_OUT) * 2 * D_OUT + cc % D_OUT,
                      1.0, 0.0).astype(jnp.float32)
    sel_r = jnp.where(rr == (cc // D_OUT) * 2 * D_OUT + D_OUT + cc % D_OUT,
                      1.0, 0.0).astype(jnp.float32)
    attn_l = jnp.dot(attn, sel_l, preferred_element_type=jnp.float32)
    attn_r = jnp.dot(attn, sel_r, preferred_element_type=jnp.float32)
    r_ids = lax.broadcasted_iota(jnp.int32, (KD, K), 0)
    c_ids = lax.broadcasted_iota(jnp.int32, (KD, K), 1)
    bmask = jnp.where(r_ids // D_OUT == c_ids, 1.0, 0.0).astype(jnp.float32)
    el = jnp.dot(fs * attn_l, bmask, preferred_element_type=jnp.float32)
    er = jnp.dot(fd * attn_r, bmask, preferred_element_type=jnp.float32)
    feat_ref[...] = fs
    el_ref[...] = el
    er_ref[...] = er


def _dense_call(x_pad, feat_rel, W_src, W_dst, W_rel):
    grid = N_PAD // _ROW_BLK
    return pl.pallas_call(
        _dense_body,
        grid=(grid,),
        in_specs=[
            pl.BlockSpec((_ROW_BLK, D_IN), lambda i: (i, 0)),
            pl.BlockSpec((1, D_REL), lambda i: (0, 0)),
            pl.BlockSpec((D_IN, KD), lambda i: (0, 0)),
            pl.BlockSpec((D_IN, KD), lambda i: (0, 0)),
            pl.BlockSpec((D_REL, 2 * KD), lambda i: (0, 0)),
        ],
        out_specs=[
            pl.BlockSpec((_ROW_BLK, KD), lambda i: (i, 0)),
            pl.BlockSpec((_ROW_BLK, K), lambda i: (i, 0)),
            pl.BlockSpec((_ROW_BLK, K), lambda i: (i, 0)),
        ],
        out_shape=[
            jax.ShapeDtypeStruct((N_PAD, KD), jnp.float32),
            jax.ShapeDtypeStruct((N_PAD, K), jnp.float32),
            jax.ShapeDtypeStruct((N_PAD, K), jnp.float32),
        ],
    )(x_pad, feat_rel.reshape(1, D_REL), W_src, W_dst, W_rel)


def kernel(x, edge_index, feat_rel, W_src, W_dst, W_rel):
    x_pad = jnp.pad(x, ((0, N_PAD - N), (0, 0)))
    feat, el, er = _dense_call(x_pad, feat_rel, W_src, W_dst, W_rel)
    feat = feat[:N]
    el = el[:N]
    er = er[:N]
    src = edge_index[0]
    dst = edge_index[1]
    e = el[src] + er[dst]
    e = jnp.where(e > 0, e, 0.2 * e)
    e_exp = jnp.exp(e)
    denom = jax.ops.segment_sum(e_exp, dst, num_segments=N)
    feat_k = feat.reshape(N, K, D_OUT)
    msg = feat_k[src] * e_exp[:, :, None]
    out = jax.ops.segment_sum(msg, dst, num_segments=N)
    den = jnp.where(denom > 0, denom, 1.0)
    out = out / den[:, :, None]
    out = jax.nn.relu(out.reshape(N, KD))
    return out


# speedup vs baseline: 66.7060x; 62.9946x over previous
import functools

import jax
import jax.numpy as jnp
from jax import lax
from jax.experimental import pallas as pl
from jax.experimental.pallas import tpu as pltpu
from jax.experimental.pallas import tpu_sc as plsc

N = 10000
E = 320000
D_IN = 128
D_REL = 64
K = 8
D_OUT = 16
KD = K * D_OUT

NUM_SC = 2
NUM_TILES = 16
ROWS_PER_TILE = 640
N_PAD = NUM_TILES * ROWS_PER_TILE
CHUNK = 64
EDGES_PER_TILE = 10240
E_PAD = NUM_SC * NUM_TILES * EDGES_PER_TILE
NUM_CHUNKS = EDGES_PER_TILE // CHUNK

_ROW_BLK = 512


def _dense_body(x_ref, fr_ref, wsrc_ref, wdst_ref, wrel_ref,
                feat_ref, el_ref, er_ref):
    x = x_ref[...]
    fs = jnp.dot(x, wsrc_ref[...], preferred_element_type=jnp.float32)
    fd = jnp.dot(x, wdst_ref[...], preferred_element_type=jnp.float32)
    attn = jnp.dot(fr_ref[...], wrel_ref[...],
                   preferred_element_type=jnp.float32)
    rr = lax.broadcasted_iota(jnp.int32, (2 * KD, KD), 0)
    cc = lax.broadcasted_iota(jnp.int32, (2 * KD, KD), 1)
    sel_l = jnp.where(rr == (cc // D_OUT) * 2 * D_OUT + cc % D_OUT,
                      1.0, 0.0).astype(jnp.float32)
    sel_r = jnp.where(rr == (cc // D_OUT) * 2 * D_OUT + D_OUT + cc % D_OUT,
                      1.0, 0.0).astype(jnp.float32)
    attn_l = jnp.dot(attn, sel_l, preferred_element_type=jnp.float32)
    attn_r = jnp.dot(attn, sel_r, preferred_element_type=jnp.float32)
    r_ids = lax.broadcasted_iota(jnp.int32, (KD, K), 0)
    c_ids = lax.broadcasted_iota(jnp.int32, (KD, K), 1)
    bmask = jnp.where(r_ids // D_OUT == c_ids, 1.0, 0.0).astype(jnp.float32)
    el = jnp.dot(fs * attn_l, bmask, preferred_element_type=jnp.float32)
    er = jnp.dot(fd * attn_r, bmask, preferred_element_type=jnp.float32)
    feat_ref[...] = fs
    el_ref[...] = el
    er_ref[...] = er


def _dense_call(x_pad, feat_rel, W_src, W_dst, W_rel):
    grid = N_PAD // _ROW_BLK
    return pl.pallas_call(
        _dense_body,
        grid=(grid,),
        in_specs=[
            pl.BlockSpec((_ROW_BLK, D_IN), lambda i: (i, 0)),
            pl.BlockSpec((1, D_REL), lambda i: (0, 0)),
            pl.BlockSpec((D_IN, KD), lambda i: (0, 0)),
            pl.BlockSpec((D_IN, KD), lambda i: (0, 0)),
            pl.BlockSpec((D_REL, 2 * KD), lambda i: (0, 0)),
        ],
        out_specs=[
            pl.BlockSpec((_ROW_BLK, KD), lambda i: (i, 0)),
            pl.BlockSpec((_ROW_BLK, K), lambda i: (i, 0)),
            pl.BlockSpec((_ROW_BLK, K), lambda i: (i, 0)),
        ],
        out_shape=[
            jax.ShapeDtypeStruct((N_PAD, KD), jnp.float32),
            jax.ShapeDtypeStruct((N_PAD, K), jnp.float32),
            jax.ShapeDtypeStruct((N_PAD, K), jnp.float32),
        ],
    )(x_pad, feat_rel.reshape(1, D_REL), W_src, W_dst, W_rel)


def _edge_body(src_hbm, dst_hbm, el_hbm, er_hbm, feat_hbm,
               num_hbm, den_hbm,
               sv0, sv1, dv0, dv1, dsc0, dsc1,
               elr0, elr1, err0, err1, wb0, wb1, ft0, ft1, mg0, mg1,
               den_sp, num_sp,
               s_sd0, s_sd1, s_g0, s_g1, s_sc0, s_sc1):
    c = lax.axis_index("c")
    s = lax.axis_index("s")
    rows0 = s * ROWS_PER_TILE
    base = c * (E_PAD // NUM_SC) + s * EDGES_PER_TILE

    SV = (sv0, sv1)
    DV = (dv0, dv1)
    DSC = (dsc0, dsc1)
    ELR = (elr0, elr1)
    ERR = (err0, err1)
    WB = (wb0, wb1)
    FT = (ft0, ft1)
    MG = (mg0, mg1)
    SSD = (s_sd0, s_sd1)
    SG = (s_g0, s_g1)
    SSC = (s_sc0, s_sc1)

    zeros16 = jnp.zeros((16,), jnp.float32)

    def _zmsg(e, carry):
        for k in range(K):
            mg0[e, pl.ds(k * 16, 16)] = zeros16
        return carry
    lax.fori_loop(0, CHUNK, _zmsg, 0)

    def _zw(g, carry):
        j = g * 16 + lax.iota(jnp.int32, 16)
        plsc.store_scatter(wb0, [j // K, j % K], zeros16)
        return carry
    lax.fori_loop(0, (CHUNK * K) // 16, _zw, 0)

    for b in range(ROWS_PER_TILE // CHUNK):
        pltpu.sync_copy(mg0, num_sp.at[pl.ds(rows0 + b * CHUNK, CHUNK)])
        pltpu.sync_copy(wb0, den_sp.at[pl.ds(rows0 + b * CHUNK, CHUNK)])

    plsc.subcore_barrier()

    def _issue_sd(p, off):
        pltpu.async_copy(src_hbm.at[pl.ds(off, CHUNK)], SV[p], SSD[p])
        pltpu.async_copy(dst_hbm.at[pl.ds(off, CHUNK)], DV[p], SSD[p])

    def _wait_sd(p):
        pltpu.make_async_copy(src_hbm.at[pl.ds(0, CHUNK)], SV[p], SSD[p]).wait()
        pltpu.make_async_copy(dst_hbm.at[pl.ds(0, CHUNK)], DV[p], SSD[p]).wait()

    def _issue_g(p):
        pltpu.async_copy(el_hbm.at[SV[p]], ELR[p], SG[p])
        pltpu.async_copy(er_hbm.at[DV[p]], ERR[p], SG[p])
        pltpu.async_copy(feat_hbm.at[SV[p]], FT[p], SG[p])

    def _wait_g(p):
        pltpu.make_async_copy(el_hbm.at[SV[p]], ELR[p], SG[p]).wait()
        pltpu.make_async_copy(er_hbm.at[DV[p]], ERR[p], SG[p]).wait()
        pltpu.make_async_copy(feat_hbm.at[SV[p]], FT[p], SG[p]).wait()

    def _issue_sc(p):
        pltpu.async_copy(MG[p], num_sp.at[DSC[p]], SSC[p], add=True)
        pltpu.async_copy(WB[p], den_sp.at[DSC[p]], SSC[p], add=True)

    def _wait_sc(p):
        pltpu.make_async_copy(MG[p], num_sp.at[DSC[p]], SSC[p]).wait()
        pltpu.make_async_copy(WB[p], den_sp.at[DSC[p]], SSC[p]).wait()

    def _compute(p):
        @plsc.parallel_loop(0, (CHUNK * K) // 16, unroll=4)
        def _wgrp(g):
            j = g * 16 + lax.iota(jnp.int32, 16)
            ei = j // K
            ki = j % K
            ev = plsc.load_gather(ELR[p], [ei, ki])
            rv = plsc.load_gather(ERR[p], [ei, ki])
            v = ev + rv
            v = jnp.where(v > 0, v, 0.2 * v)
            w = jnp.exp(v)
            plsc.store_scatter(WB[p], [ei, ki], w)

        @plsc.parallel_loop(0, CHUNK, unroll=2)
        def _mgrp(e):
            esplat = jnp.zeros((16,), jnp.int32) + e
            for k in range(K):
                kc = jnp.full((16,), k, jnp.int32)
                wv = plsc.load_gather(WB[p], [esplat, kc])
                fv = FT[p][e, pl.ds(k * D_OUT, D_OUT)]
                MG[p][e, pl.ds(k * D_OUT, D_OUT)] = fv * wv

    def _snap_dsc(p):
        for g in range(CHUNK // 16):
            DSC[p][pl.ds(g * 16, 16)] = DV[p][pl.ds(g * 16, 16)]

    _issue_sd(0, base)
    _wait_sd(0)
    _issue_g(0)
    _issue_sd(1, base + CHUNK)

    HALF = NUM_CHUNKS // 2

    def _macro(m, carry):
        i0 = 2 * m
        not_last = m <= HALF - 2
        _wait_sd(1)
        _issue_g(1)
        _wait_g(0)

        @pl.when(m >= 1)
        def _():
            _wait_sc(0)
        _snap_dsc(0)

        @pl.when(not_last)
        def _():
            _issue_sd(0, base + (i0 + 2) * CHUNK)
        _compute(0)
        _issue_sc(0)

        @pl.when(not_last)
        def _():
            _wait_sd(0)
            _issue_g(0)
        _wait_g(1)

        @pl.when(m >= 1)
        def _():
            _wait_sc(1)
        _snap_dsc(1)

        @pl.when(not_last)
        def _():
            _issue_sd(1, base + (i0 + 3) * CHUNK)
        _compute(1)
        _issue_sc(1)
        return carry
    lax.fori_loop(0, HALF, _macro, 0)

    _wait_sc(0)
    _wait_sc(1)

    plsc.subcore_barrier()

    pltpu.sync_copy(num_sp.at[pl.ds(rows0, ROWS_PER_TILE)],
                    num_hbm.at[c, pl.ds(rows0, ROWS_PER_TILE)])
    pltpu.sync_copy(den_sp.at[pl.ds(rows0, ROWS_PER_TILE)],
                    den_hbm.at[c, pl.ds(rows0, ROWS_PER_TILE)])


def _edge_call(edge_index, el, er, feat):
    mesh = plsc.VectorSubcoreMesh(core_axis_name="c", subcore_axis_name="s")
    fn = pl.kernel(
        _edge_body,
        out_type=[
            jax.ShapeDtypeStruct((NUM_SC, N_PAD, KD), jnp.float32),
            jax.ShapeDtypeStruct((NUM_SC, N_PAD, K), jnp.float32),
        ],
        mesh=mesh,
        scratch_types=(
            [pltpu.VMEM((CHUNK,), jnp.int32)] * 6
            + [pltpu.VMEM((CHUNK, K), jnp.float32)] * 6
            + [pltpu.VMEM((CHUNK, KD), jnp.float32)] * 4
            + [
                pltpu.VMEM_SHARED((N_PAD, K), jnp.float32),
                pltpu.VMEM_SHARED((N_PAD, KD), jnp.float32),
            ]
            + [pltpu.SemaphoreType.DMA] * 6
        ),
        compiler_params=pltpu.CompilerParams(needs_layout_passes=False,
                                             use_tc_tiling_on_sc=False),
    )
    npad = E_PAD - E
    src_pad = jnp.concatenate([edge_index[0],
                               jnp.zeros((npad,), jnp.int32)])
    dst_pad = jnp.concatenate([edge_index[1],
                               jnp.full((npad,), N_PAD - 1, jnp.int32)])
    return fn(src_pad, dst_pad, el, er, feat)


def _norm_body(num_ref, den_ref, out_ref):
    num = num_ref[0] + num_ref[1]
    den = den_ref[0] + den_ref[1]
    r_ids = lax.broadcasted_iota(jnp.int32, (K, KD), 0)
    c_ids = lax.broadcasted_iota(jnp.int32, (K, KD), 1)
    expand = jnp.where(c_ids // D_OUT == r_ids, 1.0, 0.0).astype(jnp.float32)
    den128 = jnp.dot(den, expand, preferred_element_type=jnp.float32)
    safe = jnp.where(den128 > 0, den128, 1.0)
    out_ref[...] = jnp.maximum(num, 0.0) / safe


def _norm_call(num2, den2):
    blk = 512
    grid = N_PAD // blk
    return pl.pallas_call(
        _norm_body,
        grid=(grid,),
        in_specs=[
            pl.BlockSpec((NUM_SC, blk, KD), lambda i: (0, i, 0)),
            pl.BlockSpec((NUM_SC, blk, K), lambda i: (0, i, 0)),
        ],
        out_specs=pl.BlockSpec((blk, KD), lambda i: (i, 0)),
        out_shape=jax.ShapeDtypeStruct((N_PAD, KD), jnp.float32),
    )(num2, den2)


def kernel(x, edge_index, feat_rel, W_src, W_dst, W_rel):
    x_pad = jnp.pad(x, ((0, N_PAD - N), (0, 0)))
    feat, el, er = _dense_call(x_pad, feat_rel, W_src, W_dst, W_rel)
    num2, den2 = _edge_call(edge_index, el, er, feat)
    out = _norm_call(num2, den2)
    return out[:N]
